# baseline (speedup 1.0000x reference)
import functools
import math

import jax
import jax.numpy as jnp
from jax.experimental import pallas as pl
from jax.experimental.pallas import tpu as pltpu
from jax.experimental.pallas import tpu_sc as plsc

_NN = 16
_GW = 128
_P = 200


def _gather_body(ctab_hbm, ftab_hbm, idx_hbm, oc_hbm, of_hbm, *, n_rows):
    def body(i_vmem, oc_vmem, of_vmem):
        pltpu.sync_copy(ctab_hbm.at[i_vmem.at[0]], oc_vmem)
        pltpu.sync_copy(ftab_hbm.at[i_vmem.at[0]], of_vmem)

    pltpu.emit_pipeline(
        body,
        grid=(n_rows // _GW,),
        in_specs=[pl.BlockSpec((1, _GW), lambda i: (0, i))],
        out_specs=[pl.BlockSpec((_GW, 16), lambda i: (i, 0)),
                   pl.BlockSpec((_GW, 32), lambda i: (i, 0))],
        core_axis_name=("c", "s"),
        dimension_semantics=(pltpu.PARALLEL,),
    )(idx_hbm, oc_hbm, of_hbm)


def _round_bf16(v):
    return v.astype(jnp.bfloat16).astype(jnp.float32)


def _paiconv_block(c_ref, f_ref, bm_ref, mlpw_ref, mlpb_ref,
                   kt_ref, op_ref, w2t_ref, smat_ref, cb_ref, o_ref,
                   *, npts, nn):
    P, K = npts, nn
    R = P * K
    two_pi = 2.0 * math.pi
    c = c_ref[...]
    c3 = c.reshape(P, K, 16)
    x0 = c3[:, 0:1, :]
    xr = c3 - x0
    xr2 = xr * xr
    dis = jnp.sqrt(xr2[:, :, 0:1] + xr2[:, :, 1:2] + xr2[:, :, 2:3])

    bm = _round_bf16(bm_ref[...])
    xf = (_round_bf16(two_pi * x0[:, :, 0:1]) * bm[0:1, :][None]
          + _round_bf16(two_pi * x0[:, :, 1:2]) * bm[1:2, :][None]
          + _round_bf16(two_pi * x0[:, :, 2:3]) * bm[2:3, :][None]
          + _round_bf16(two_pi * xr[:, :, 0:1]) * bm[3:4, :][None]
          + _round_bf16(two_pi * xr[:, :, 1:2]) * bm[4:5, :][None]
          + _round_bf16(two_pi * xr[:, :, 2:3]) * bm[5:6, :][None]
          + _round_bf16(two_pi * dis) * bm[6:7, :][None])
    xf = xf.reshape(R, 32)
    sc = jnp.concatenate([jnp.sin(xf), jnp.cos(xf)], axis=-1)
    mlpout = (jnp.dot(sc, mlpw_ref[...], preferred_element_type=jnp.float32)
              + mlpb_ref[0:1, :])

    g = jnp.concatenate([f_ref[...], mlpout], axis=-1)
    e = jnp.dot(g, w2t_ref[...], preferred_element_type=jnp.float32)

    kt = _round_bf16(kt_ref[...])
    praw = (_round_bf16(xr[:, :, 0:1]) * kt[0:1, :][None]
            + _round_bf16(xr[:, :, 1:2]) * kt[1:2, :][None]
            + _round_bf16(xr[:, :, 2:3]) * kt[2:3, :][None]) + op_ref[...][None]
    p = jnp.maximum(praw, 0.0)
    p = p / (jnp.sum(p, axis=1, keepdims=True) + 1e-6)
    p = p * p
    p = p / (jnp.sum(p, axis=1, keepdims=True) + 1e-6)
    p = jnp.where(p > 0.1, p, jnp.zeros_like(p))
    ptile = pltpu.repeat(p.reshape(R, K), 32, axis=1)

    y = jnp.dot(e * ptile, smat_ref[...], preferred_element_type=jnp.float32)
    out = jnp.sum(y.reshape(P, K, 32), axis=1) + cb_ref[0:1, :]
    o_ref[...] = out


def kernel(x, feature, neigh_indexs, Bmat, kernels, mlp_w, mlp_b, conv_w, conv_b):
    k = _NN
    bsize, num_feat, num_pts = feature.shape
    out_c = conv_w.shape[0]
    M = bsize * num_pts
    n_rows = M * k

    xp = jnp.transpose(x, (0, 2, 1)).reshape(M, 3).astype(jnp.float32)
    ctab = jnp.pad(xp, ((0, 0), (0, 13)))
    ftab = jnp.transpose(feature, (0, 2, 1)).reshape(M, num_feat).astype(jnp.float32)

    neigh = neigh_indexs[:, :, :k].astype(jnp.int32)
    base = (jnp.arange(bsize, dtype=jnp.int32) * num_pts)[:, None, None]
    idx = (neigh + base).reshape(1, n_rows)

    mesh = plsc.VectorSubcoreMesh(core_axis_name="c", subcore_axis_name="s")
    gather = pl.kernel(
        out_type=(jax.ShapeDtypeStruct((n_rows, 16), jnp.float32),
                  jax.ShapeDtypeStruct((n_rows, num_feat), jnp.float32)),
        mesh=mesh,
        compiler_params=pltpu.CompilerParams(use_tc_tiling_on_sc=False),
    )(functools.partial(_gather_body, n_rows=n_rows))
    crows, frows = gather(ctab, ftab, idx)

    bmp = jnp.zeros((8, 32), jnp.float32).at[0:7].set(Bmat.astype(jnp.float32))
    mlpwT = mlp_w.T.astype(jnp.float32)
    mlpbp = jnp.zeros((8, 32), jnp.float32).at[0].set(mlp_b)
    ktp = jnp.zeros((8, 16), jnp.float32).at[0:3].set(kernels.astype(jnp.float32))
    onepad = jnp.zeros((16, 16), jnp.float32).at[0, 0].set(1.0)

    ng = 4
    width = 2 * num_feat // ng
    c_ar = jnp.arange(2 * num_feat)
    cperm = (c_ar % width) * ng + c_ar // width
    cw3 = conv_w.reshape(out_c, 2 * num_feat, k)
    w2t = jnp.transpose(cw3[:, cperm, :], (1, 0, 2)).reshape(
        2 * num_feat, out_c * k).astype(jnp.float32)
    smat = (jnp.arange(out_c * k)[:, None] // k
            == jnp.arange(out_c)[None, :]).astype(jnp.float32)
    cbp = jnp.zeros((8, 32), jnp.float32).at[0].set(conv_b)

    nblocks = M // _P
    R = _P * k
    grid_spec = pl.GridSpec(
        grid=(nblocks,),
        in_specs=[
            pl.BlockSpec((R, 16), lambda i: (i, 0)),
            pl.BlockSpec((R, 32), lambda i: (i, 0)),
            pl.BlockSpec((8, 32), lambda i: (0, 0)),
            pl.BlockSpec((64, 32), lambda i: (0, 0)),
            pl.BlockSpec((8, 32), lambda i: (0, 0)),
            pl.BlockSpec((8, 16), lambda i: (0, 0)),
            pl.BlockSpec((16, 16), lambda i: (0, 0)),
            pl.BlockSpec((64, 512), lambda i: (0, 0)),
            pl.BlockSpec((512, 32), lambda i: (0, 0)),
            pl.BlockSpec((8, 32), lambda i: (0, 0)),
        ],
        out_specs=pl.BlockSpec((_P, 32), lambda i: (i, 0)),
    )
    out2 = pl.pallas_call(
        functools.partial(_paiconv_block, npts=_P, nn=k),
        grid_spec=grid_spec,
        out_shape=jax.ShapeDtypeStruct((M, 32), jnp.float32),
    )(crows, frows, bmp, mlpwT, mlpbp, ktp, onepad, w2t, smat, cbp)

    out = out2.reshape(bsize, num_pts, out_c)
    return jnp.transpose(out, (0, 2, 1))
